# Initial kernel scaffold; baseline (speedup 1.0000x reference)
import jax
import jax.numpy as jnp
from jax.experimental import pallas as pl


def kernel(x, edge_index, W1, b1, W2, b2, Wfc, bfc):
    raise NotImplementedError("write your pallas kernel here")


# speedup vs baseline: 19.4785x; 19.4785x over previous
import functools

import jax
import jax.numpy as jnp
from jax import lax
from jax.experimental import pallas as pl
from jax.experimental.pallas import tpu as pltpu
from jax.experimental.pallas import tpu_sc as plsc

N = 10000
E = 160000
NP = 10240
EP = 163840
CHUNK = 128
CPT = (EP // 32) // CHUNK
ROWS_PT = NP // 16

_mesh = plsc.VectorSubcoreMesh(core_axis_name="c", subcore_axis_name="s")
_f32 = jnp.float32
_sc_params = pltpu.CompilerParams(use_tc_tiling_on_sc=False)


@functools.partial(
    pl.kernel,
    out_type=jax.ShapeDtypeStruct((2, NP, 16), _f32),
    mesh=_mesh,
    scratch_types=[
        pltpu.VMEM((CPT, CHUNK), jnp.int32),
        pltpu.VMEM((CHUNK, 16), _f32),
        pltpu.VMEM_SHARED((NP, 16), _f32),
        pltpu.SemaphoreType.DMA,
    ],
    compiler_params=_sc_params,
)
def _deg_sc(dst_hbm, ones_hbm, zeros_hbm, out_hbm, idx_v, ones_v, acc, sem):
    cid = lax.axis_index("c")
    sid = lax.axis_index("s")
    tid = cid * 16 + sid
    pltpu.sync_copy(dst_hbm.at[pl.ds(tid * CPT, CPT)], idx_v)
    pltpu.sync_copy(ones_hbm, ones_v)
    pltpu.sync_copy(zeros_hbm, acc.at[pl.ds(sid * ROWS_PT, ROWS_PT)])
    plsc.subcore_barrier()

    @pl.loop(0, CPT)
    def _(j):
        pltpu.sync_copy(ones_v, acc.at[idx_v.at[j]], add=True)

    plsc.subcore_barrier()
    pltpu.sync_copy(
        acc.at[pl.ds(sid * ROWS_PT, ROWS_PT)],
        out_hbm.at[cid, pl.ds(sid * ROWS_PT, ROWS_PT)],
    )


def _make_agg(d):
    @functools.partial(
        pl.kernel,
        out_type=jax.ShapeDtypeStruct((2, NP, d), _f32),
        mesh=_mesh,
        scratch_types=[
            pltpu.VMEM((CPT, CHUNK), jnp.int32),
            pltpu.VMEM((CPT, CHUNK), jnp.int32),
            pltpu.VMEM((CHUNK, d), _f32),
            pltpu.VMEM((CHUNK, d), _f32),
            pltpu.VMEM_SHARED((NP, d), _f32),
            pltpu.SemaphoreType.DMA,
            pltpu.SemaphoreType.DMA,
        ],
        compiler_params=_sc_params,
    )
    def _agg(y_hbm, src_hbm, dst_hbm, zeros_hbm, out_hbm,
             srcv, dstv, bufa, bufb, acc, sema, semb):
        cid = lax.axis_index("c")
        sid = lax.axis_index("s")
        tid = cid * 16 + sid
        pltpu.sync_copy(src_hbm.at[pl.ds(tid * CPT, CPT)], srcv)
        pltpu.sync_copy(dst_hbm.at[pl.ds(tid * CPT, CPT)], dstv)
        pltpu.sync_copy(zeros_hbm, acc.at[pl.ds(sid * ROWS_PT, ROWS_PT)])
        plsc.subcore_barrier()

        pltpu.async_copy(y_hbm.at[srcv.at[0]], bufa, sema)

        @pl.loop(0, CPT, step=2)
        def _(j):
            pltpu.async_copy(y_hbm.at[srcv.at[j + 1]], bufb, semb)
            pltpu.make_async_copy(y_hbm.at[srcv.at[j]], bufa, sema).wait()
            pltpu.sync_copy(bufa, acc.at[dstv.at[j]], add=True)

            @pl.when(j + 2 < CPT)
            def _():
                pltpu.async_copy(y_hbm.at[srcv.at[j + 2]], bufa, sema)

            pltpu.make_async_copy(y_hbm.at[srcv.at[j + 1]], bufb, semb).wait()
            pltpu.sync_copy(bufb, acc.at[dstv.at[j + 1]], add=True)

        plsc.subcore_barrier()
        pltpu.sync_copy(
            acc.at[pl.ds(sid * ROWS_PT, ROWS_PT)],
            out_hbm.at[cid, pl.ds(sid * ROWS_PT, ROWS_PT)],
        )

    return _agg


_agg64 = _make_agg(64)
_agg32 = _make_agg(32)


_BLK = 256
_GRID = NP // _BLK


def _dis_invdeg(dp_ref):
    deg = 1.0 + dp_ref[0, :, 0:1] + dp_ref[1, :, 0:1]
    dis = 1.0 / jnp.sqrt(deg)
    return dis, dis * dis


def _dense1_body(x_ref, w_ref, dp_ref, y_ref, s_ref):
    xw = jnp.dot(x_ref[...], w_ref[...], preferred_element_type=_f32)
    dis, invd = _dis_invdeg(dp_ref)
    y_ref[...] = xw * dis
    s_ref[...] = xw * invd


def _dense1(x, w1, dp):
    return pl.pallas_call(
        _dense1_body,
        grid=(_GRID,),
        in_specs=[
            pl.BlockSpec((_BLK, 256), lambda i: (i, 0)),
            pl.BlockSpec((256, 64), lambda i: (0, 0)),
            pl.BlockSpec((2, _BLK, 16), lambda i: (0, i, 0)),
        ],
        out_specs=[
            pl.BlockSpec((_BLK, 64), lambda i: (i, 0)),
            pl.BlockSpec((_BLK, 64), lambda i: (i, 0)),
        ],
        out_shape=[
            jax.ShapeDtypeStruct((N, 64), _f32),
            jax.ShapeDtypeStruct((N, 64), _f32),
        ],
    )(x, w1, dp)


def _dense2_body(ap_ref, s1_ref, dp_ref, w_ref, b_ref, y_ref, s_ref):
    dis, invd = _dis_invdeg(dp_ref)
    agg = ap_ref[0] + ap_ref[1]
    h = jnp.maximum(agg * dis + s1_ref[...] + b_ref[...], 0.0)
    xw = jnp.dot(h, w_ref[...], preferred_element_type=_f32)
    y_ref[...] = xw * dis
    s_ref[...] = xw * invd


def _dense2(ap, s1, dp, w2, b1):
    return pl.pallas_call(
        _dense2_body,
        grid=(_GRID,),
        in_specs=[
            pl.BlockSpec((2, _BLK, 64), lambda i: (0, i, 0)),
            pl.BlockSpec((_BLK, 64), lambda i: (i, 0)),
            pl.BlockSpec((2, _BLK, 16), lambda i: (0, i, 0)),
            pl.BlockSpec((64, 32), lambda i: (0, 0)),
            pl.BlockSpec((1, 64), lambda i: (0, 0)),
        ],
        out_specs=[
            pl.BlockSpec((_BLK, 32), lambda i: (i, 0)),
            pl.BlockSpec((_BLK, 32), lambda i: (i, 0)),
        ],
        out_shape=[
            jax.ShapeDtypeStruct((N, 32), _f32),
            jax.ShapeDtypeStruct((N, 32), _f32),
        ],
    )(ap, s1, dp, w2, b1)


def _dense3_body(ap_ref, s2_ref, dp_ref, wfc_ref, b2_ref, bfc_ref, o_ref):
    dis, _ = _dis_invdeg(dp_ref)
    agg = ap_ref[0] + ap_ref[1]
    h = jnp.maximum(agg * dis + s2_ref[...] + b2_ref[...], 0.0)
    o_ref[...] = jnp.sum(h * wfc_ref[...], axis=1, keepdims=True) + bfc_ref[...]


def _dense3(ap, s2, dp, wfc_row, b2, bfc):
    return pl.pallas_call(
        _dense3_body,
        grid=(_GRID,),
        in_specs=[
            pl.BlockSpec((2, _BLK, 32), lambda i: (0, i, 0)),
            pl.BlockSpec((_BLK, 32), lambda i: (i, 0)),
            pl.BlockSpec((2, _BLK, 16), lambda i: (0, i, 0)),
            pl.BlockSpec((1, 32), lambda i: (0, 0)),
            pl.BlockSpec((1, 32), lambda i: (0, 0)),
            pl.BlockSpec((1, 1), lambda i: (0, 0)),
        ],
        out_specs=pl.BlockSpec((_BLK, 1), lambda i: (i, 0)),
        out_shape=jax.ShapeDtypeStruct((N, 1), _f32),
    )(ap, s2, dp, wfc_row, b2, bfc)


def kernel(x, edge_index, W1, b1, W2, b2, Wfc, bfc):
    src = edge_index[0].astype(jnp.int32)
    dst = edge_index[1].astype(jnp.int32)
    pad = jnp.arange(EP - E, dtype=jnp.int32)
    srcp = jnp.concatenate([src, pad % N]).reshape(EP // CHUNK, CHUNK)
    dstp = jnp.concatenate([dst, N + pad % (NP - N)]).reshape(EP // CHUNK, CHUNK)

    ones16 = jnp.ones((CHUNK, 16), _f32)
    z16 = jnp.zeros((ROWS_PT, 16), _f32)
    z64 = jnp.zeros((ROWS_PT, 64), _f32)
    z32 = jnp.zeros((ROWS_PT, 32), _f32)

    dp = _deg_sc(dstp, ones16, z16)
    y1, s1 = _dense1(x, W1, dp)
    a1 = _agg64(y1, srcp, dstp, z64)
    y2, s2 = _dense2(a1, s1, dp, W2, b1.reshape(1, 64))
    a2 = _agg32(y2, srcp, dstp, z32)
    return _dense3(a2, s2, dp, Wfc.reshape(1, 32),
                   b2.reshape(1, 32), bfc.reshape(1, 1))


# speedup vs baseline: 24.2551x; 1.2452x over previous
import functools

import jax
import jax.numpy as jnp
from jax import lax
from jax.experimental import pallas as pl
from jax.experimental.pallas import tpu as pltpu
from jax.experimental.pallas import tpu_sc as plsc

N = 10000
E = 160000
NP = 10240
EP = 163840
CHUNK = 128
CPT = (EP // 32) // CHUNK
ROWS_PT = NP // 16

_mesh = plsc.VectorSubcoreMesh(core_axis_name="c", subcore_axis_name="s")
_f32 = jnp.float32
_sc_params = pltpu.CompilerParams(use_tc_tiling_on_sc=False)


@functools.partial(
    pl.kernel,
    out_type=jax.ShapeDtypeStruct((2, NP, 16), _f32),
    mesh=_mesh,
    scratch_types=[
        pltpu.VMEM((CPT, CHUNK), jnp.int32),
        pltpu.VMEM((CHUNK, 16), _f32),
        pltpu.VMEM_SHARED((NP, 16), _f32),
        pltpu.SemaphoreType.DMA,
    ],
    compiler_params=_sc_params,
)
def _deg_sc(dst_hbm, ones_hbm, zeros_hbm, out_hbm, idx_v, ones_v, acc, sem):
    cid = lax.axis_index("c")
    sid = lax.axis_index("s")
    tid = cid * 16 + sid
    pltpu.sync_copy(dst_hbm.at[pl.ds(tid * CPT, CPT)], idx_v)
    pltpu.sync_copy(ones_hbm, ones_v)
    pltpu.sync_copy(zeros_hbm, acc.at[pl.ds(sid * ROWS_PT, ROWS_PT)])
    plsc.subcore_barrier()

    @pl.loop(0, CPT)
    def _(j):
        pltpu.sync_copy(ones_v, acc.at[idx_v.at[j]], add=True)

    plsc.subcore_barrier()
    pltpu.sync_copy(
        acc.at[pl.ds(sid * ROWS_PT, ROWS_PT)],
        out_hbm.at[cid, pl.ds(sid * ROWS_PT, ROWS_PT)],
    )


def _make_agg(d):
    @functools.partial(
        pl.kernel,
        out_type=jax.ShapeDtypeStruct((2, NP, d), _f32),
        mesh=_mesh,
        scratch_types=[
            pltpu.VMEM((CPT, CHUNK), jnp.int32),
            pltpu.VMEM((CPT, CHUNK), jnp.int32),
            pltpu.VMEM((CHUNK, d), _f32),
            pltpu.VMEM((CHUNK, d), _f32),
            pltpu.VMEM_SHARED((NP, d), _f32),
            pltpu.SemaphoreType.DMA,
            pltpu.SemaphoreType.DMA,
        ],
        compiler_params=_sc_params,
    )
    def _agg(y_hbm, src_hbm, dst_hbm, zeros_hbm, out_hbm,
             srcv, dstv, bufa, bufb, acc, sema, semb):
        cid = lax.axis_index("c")
        sid = lax.axis_index("s")
        tid = cid * 16 + sid
        pltpu.sync_copy(src_hbm.at[pl.ds(tid * CPT, CPT)], srcv)
        pltpu.sync_copy(dst_hbm.at[pl.ds(tid * CPT, CPT)], dstv)
        pltpu.sync_copy(zeros_hbm, acc.at[pl.ds(sid * ROWS_PT, ROWS_PT)])
        plsc.subcore_barrier()

        pltpu.async_copy(y_hbm.at[srcv.at[0]], bufa, sema)

        @pl.loop(0, CPT, step=2)
        def _(j):
            pltpu.async_copy(y_hbm.at[srcv.at[j + 1]], bufb, semb)
            pltpu.make_async_copy(y_hbm.at[srcv.at[j]], bufa, sema).wait()
            pltpu.sync_copy(bufa, acc.at[dstv.at[j]], add=True)

            @pl.when(j + 2 < CPT)
            def _():
                pltpu.async_copy(y_hbm.at[srcv.at[j + 2]], bufa, sema)

            pltpu.make_async_copy(y_hbm.at[srcv.at[j + 1]], bufb, semb).wait()
            pltpu.sync_copy(bufb, acc.at[dstv.at[j + 1]], add=True)

        plsc.subcore_barrier()
        pltpu.sync_copy(
            acc.at[pl.ds(sid * ROWS_PT, ROWS_PT)],
            out_hbm.at[cid, pl.ds(sid * ROWS_PT, ROWS_PT)],
        )

    return _agg


_agg64 = _make_agg(64)
_agg32 = _make_agg(32)


_BLK = 1024
_GRID = NP // _BLK


def _dis_invdeg(dp_ref):
    deg = 1.0 + dp_ref[0, :, 0:1] + dp_ref[1, :, 0:1]
    dis = 1.0 / jnp.sqrt(deg)
    return dis, dis * dis


def _dense1_body(x_ref, w_ref, dp_ref, y_ref, s_ref):
    xw = jnp.dot(x_ref[...], w_ref[...], preferred_element_type=_f32)
    dis, invd = _dis_invdeg(dp_ref)
    y_ref[...] = xw * dis
    s_ref[...] = xw * invd


def _dense1(x, w1, dp):
    return pl.pallas_call(
        _dense1_body,
        grid=(_GRID,),
        in_specs=[
            pl.BlockSpec((_BLK, 256), lambda i: (i, 0)),
            pl.BlockSpec((256, 64), lambda i: (0, 0)),
            pl.BlockSpec((2, _BLK, 16), lambda i: (0, i, 0)),
        ],
        out_specs=[
            pl.BlockSpec((_BLK, 64), lambda i: (i, 0)),
            pl.BlockSpec((_BLK, 64), lambda i: (i, 0)),
        ],
        out_shape=[
            jax.ShapeDtypeStruct((N, 64), _f32),
            jax.ShapeDtypeStruct((N, 64), _f32),
        ],
    )(x, w1, dp)


def _dense2_body(ap_ref, s1_ref, dp_ref, w_ref, b_ref, y_ref, s_ref):
    dis, invd = _dis_invdeg(dp_ref)
    agg = ap_ref[0] + ap_ref[1]
    h = jnp.maximum(agg * dis + s1_ref[...] + b_ref[...], 0.0)
    xw = jnp.dot(h, w_ref[...], preferred_element_type=_f32)
    y_ref[...] = xw * dis
    s_ref[...] = xw * invd


def _dense2(ap, s1, dp, w2, b1):
    return pl.pallas_call(
        _dense2_body,
        grid=(_GRID,),
        in_specs=[
            pl.BlockSpec((2, _BLK, 64), lambda i: (0, i, 0)),
            pl.BlockSpec((_BLK, 64), lambda i: (i, 0)),
            pl.BlockSpec((2, _BLK, 16), lambda i: (0, i, 0)),
            pl.BlockSpec((64, 32), lambda i: (0, 0)),
            pl.BlockSpec((1, 64), lambda i: (0, 0)),
        ],
        out_specs=[
            pl.BlockSpec((_BLK, 32), lambda i: (i, 0)),
            pl.BlockSpec((_BLK, 32), lambda i: (i, 0)),
        ],
        out_shape=[
            jax.ShapeDtypeStruct((N, 32), _f32),
            jax.ShapeDtypeStruct((N, 32), _f32),
        ],
    )(ap, s1, dp, w2, b1)


def _dense3_body(ap_ref, s2_ref, dp_ref, wfc_ref, b2_ref, bfc_ref, o_ref):
    dis, _ = _dis_invdeg(dp_ref)
    agg = ap_ref[0] + ap_ref[1]
    h = jnp.maximum(agg * dis + s2_ref[...] + b2_ref[...], 0.0)
    o_ref[...] = jnp.sum(h * wfc_ref[...], axis=1, keepdims=True) + bfc_ref[...]


def _dense3(ap, s2, dp, wfc_row, b2, bfc):
    return pl.pallas_call(
        _dense3_body,
        grid=(_GRID,),
        in_specs=[
            pl.BlockSpec((2, _BLK, 32), lambda i: (0, i, 0)),
            pl.BlockSpec((_BLK, 32), lambda i: (i, 0)),
            pl.BlockSpec((2, _BLK, 16), lambda i: (0, i, 0)),
            pl.BlockSpec((1, 32), lambda i: (0, 0)),
            pl.BlockSpec((1, 32), lambda i: (0, 0)),
            pl.BlockSpec((1, 1), lambda i: (0, 0)),
        ],
        out_specs=pl.BlockSpec((_BLK, 1), lambda i: (i, 0)),
        out_shape=jax.ShapeDtypeStruct((N, 1), _f32),
    )(ap, s2, dp, wfc_row, b2, bfc)


def kernel(x, edge_index, W1, b1, W2, b2, Wfc, bfc):
    src = edge_index[0].astype(jnp.int32)
    dst = edge_index[1].astype(jnp.int32)
    pad = jnp.arange(EP - E, dtype=jnp.int32)
    srcp = jnp.concatenate([src, pad % N]).reshape(EP // CHUNK, CHUNK)
    dstp = jnp.concatenate([dst, N + pad % (NP - N)]).reshape(EP // CHUNK, CHUNK)

    ones16 = jnp.ones((CHUNK, 16), _f32)
    z16 = jnp.zeros((ROWS_PT, 16), _f32)
    z64 = jnp.zeros((ROWS_PT, 64), _f32)
    z32 = jnp.zeros((ROWS_PT, 32), _f32)

    dp = _deg_sc(dstp, ones16, z16)
    y1, s1 = _dense1(x, W1, dp)
    a1 = _agg64(y1, srcp, dstp, z64)
    y2, s2 = _dense2(a1, s1, dp, W2, b1.reshape(1, 64))
    a2 = _agg32(y2, srcp, dstp, z32)
    return _dense3(a2, s2, dp, Wfc.reshape(1, 32),
                   b2.reshape(1, 32), bfc.reshape(1, 1))


# speedup vs baseline: 24.8250x; 1.0235x over previous
import functools

import jax
import jax.numpy as jnp
from jax import lax
from jax.experimental import pallas as pl
from jax.experimental.pallas import tpu as pltpu
from jax.experimental.pallas import tpu_sc as plsc

N = 10000
E = 160000
NP = 10240
EP = 163840
CHUNK = 128
CPT = (EP // 32) // CHUNK
ROWS_PT = NP // 16

_mesh = plsc.VectorSubcoreMesh(core_axis_name="c", subcore_axis_name="s")
_f32 = jnp.float32
_sc_params = pltpu.CompilerParams(use_tc_tiling_on_sc=False)


@functools.partial(
    pl.kernel,
    out_type=jax.ShapeDtypeStruct((2, NP, 16), _f32),
    mesh=_mesh,
    scratch_types=[
        pltpu.VMEM((CPT, CHUNK), jnp.int32),
        pltpu.VMEM((CHUNK, 16), _f32),
        pltpu.VMEM_SHARED((NP, 16), _f32),
        pltpu.SemaphoreType.DMA,
    ],
    compiler_params=_sc_params,
)
def _deg_sc(dst_hbm, ones_hbm, zeros_hbm, out_hbm, idx_v, ones_v, acc, sem):
    cid = lax.axis_index("c")
    sid = lax.axis_index("s")
    tid = cid * 16 + sid
    pltpu.sync_copy(dst_hbm.at[pl.ds(tid * CPT, CPT)], idx_v)
    pltpu.sync_copy(ones_hbm, ones_v)
    pltpu.sync_copy(zeros_hbm, acc.at[pl.ds(sid * ROWS_PT, ROWS_PT)])
    plsc.subcore_barrier()

    @pl.loop(0, CPT)
    def _(j):
        pltpu.sync_copy(ones_v, acc.at[idx_v.at[j]], add=True)

    plsc.subcore_barrier()
    pltpu.sync_copy(
        acc.at[pl.ds(sid * ROWS_PT, ROWS_PT)],
        out_hbm.at[cid, pl.ds(sid * ROWS_PT, ROWS_PT)],
    )


def _make_agg(d):
    @functools.partial(
        pl.kernel,
        out_type=jax.ShapeDtypeStruct((2, NP, d), _f32),
        mesh=_mesh,
        scratch_types=[
            pltpu.VMEM((CPT, CHUNK), jnp.int32),
            pltpu.VMEM((CPT, CHUNK), jnp.int32),
            pltpu.VMEM((CHUNK, d), _f32),
            pltpu.VMEM((CHUNK, d), _f32),
            pltpu.VMEM_SHARED((NP, d), _f32),
            pltpu.SemaphoreType.DMA,
            pltpu.SemaphoreType.DMA,
        ],
        compiler_params=_sc_params,
    )
    def _agg(y_hbm, src_hbm, dst_hbm, zeros_hbm, out_hbm,
             srcv, dstv, bufa, bufb, acc, sema, semb):
        cid = lax.axis_index("c")
        sid = lax.axis_index("s")
        tid = cid * 16 + sid
        pltpu.sync_copy(src_hbm.at[pl.ds(tid * CPT, CPT)], srcv)
        pltpu.sync_copy(dst_hbm.at[pl.ds(tid * CPT, CPT)], dstv)
        pltpu.sync_copy(zeros_hbm, acc.at[pl.ds(sid * ROWS_PT, ROWS_PT)])
        plsc.subcore_barrier()

        pltpu.async_copy(y_hbm.at[srcv.at[0]], bufa, sema)

        @pl.loop(0, CPT, step=2)
        def _(j):
            pltpu.async_copy(y_hbm.at[srcv.at[j + 1]], bufb, semb)
            pltpu.make_async_copy(y_hbm.at[srcv.at[j]], bufa, sema).wait()
            pltpu.sync_copy(bufa, acc.at[dstv.at[j]], add=True)

            @pl.when(j + 2 < CPT)
            def _():
                pltpu.async_copy(y_hbm.at[srcv.at[j + 2]], bufa, sema)

            pltpu.make_async_copy(y_hbm.at[srcv.at[j + 1]], bufb, semb).wait()
            pltpu.sync_copy(bufb, acc.at[dstv.at[j + 1]], add=True)

        plsc.subcore_barrier()
        pltpu.sync_copy(
            acc.at[pl.ds(sid * ROWS_PT, ROWS_PT)],
            out_hbm.at[cid, pl.ds(sid * ROWS_PT, ROWS_PT)],
        )

    return _agg


_agg64 = _make_agg(64)
_agg32 = _make_agg(32)


_BLK = 2048
_GRID = NP // _BLK


def _dis_invdeg(dp_ref):
    deg = 1.0 + dp_ref[0, :, 0:1] + dp_ref[1, :, 0:1]
    dis = 1.0 / jnp.sqrt(deg)
    return dis, dis * dis


def _dense1_body(x_ref, w_ref, dp_ref, y_ref, s_ref):
    xw = jnp.dot(x_ref[...], w_ref[...], preferred_element_type=_f32)
    dis, invd = _dis_invdeg(dp_ref)
    y_ref[...] = xw * dis
    s_ref[...] = xw * invd


def _dense1(x, w1, dp):
    return pl.pallas_call(
        _dense1_body,
        grid=(_GRID,),
        in_specs=[
            pl.BlockSpec((_BLK, 256), lambda i: (i, 0)),
            pl.BlockSpec((256, 64), lambda i: (0, 0)),
            pl.BlockSpec((2, _BLK, 16), lambda i: (0, i, 0)),
        ],
        out_specs=[
            pl.BlockSpec((_BLK, 64), lambda i: (i, 0)),
            pl.BlockSpec((_BLK, 64), lambda i: (i, 0)),
        ],
        out_shape=[
            jax.ShapeDtypeStruct((N, 64), _f32),
            jax.ShapeDtypeStruct((N, 64), _f32),
        ],
    )(x, w1, dp)


def _dense2_body(ap_ref, s1_ref, dp_ref, w_ref, b_ref, y_ref, s_ref):
    dis, invd = _dis_invdeg(dp_ref)
    agg = ap_ref[0] + ap_ref[1]
    h = jnp.maximum(agg * dis + s1_ref[...] + b_ref[...], 0.0)
    xw = jnp.dot(h, w_ref[...], preferred_element_type=_f32)
    y_ref[...] = xw * dis
    s_ref[...] = xw * invd


def _dense2(ap, s1, dp, w2, b1):
    return pl.pallas_call(
        _dense2_body,
        grid=(_GRID,),
        in_specs=[
            pl.BlockSpec((2, _BLK, 64), lambda i: (0, i, 0)),
            pl.BlockSpec((_BLK, 64), lambda i: (i, 0)),
            pl.BlockSpec((2, _BLK, 16), lambda i: (0, i, 0)),
            pl.BlockSpec((64, 32), lambda i: (0, 0)),
            pl.BlockSpec((1, 64), lambda i: (0, 0)),
        ],
        out_specs=[
            pl.BlockSpec((_BLK, 32), lambda i: (i, 0)),
            pl.BlockSpec((_BLK, 32), lambda i: (i, 0)),
        ],
        out_shape=[
            jax.ShapeDtypeStruct((N, 32), _f32),
            jax.ShapeDtypeStruct((N, 32), _f32),
        ],
    )(ap, s1, dp, w2, b1)


def _dense3_body(ap_ref, s2_ref, dp_ref, wfc_ref, b2_ref, bfc_ref, o_ref):
    dis, _ = _dis_invdeg(dp_ref)
    agg = ap_ref[0] + ap_ref[1]
    h = jnp.maximum(agg * dis + s2_ref[...] + b2_ref[...], 0.0)
    o_ref[...] = jnp.sum(h * wfc_ref[...], axis=1, keepdims=True) + bfc_ref[...]


def _dense3(ap, s2, dp, wfc_row, b2, bfc):
    return pl.pallas_call(
        _dense3_body,
        grid=(_GRID,),
        in_specs=[
            pl.BlockSpec((2, _BLK, 32), lambda i: (0, i, 0)),
            pl.BlockSpec((_BLK, 32), lambda i: (i, 0)),
            pl.BlockSpec((2, _BLK, 16), lambda i: (0, i, 0)),
            pl.BlockSpec((1, 32), lambda i: (0, 0)),
            pl.BlockSpec((1, 32), lambda i: (0, 0)),
            pl.BlockSpec((1, 1), lambda i: (0, 0)),
        ],
        out_specs=pl.BlockSpec((_BLK, 1), lambda i: (i, 0)),
        out_shape=jax.ShapeDtypeStruct((N, 1), _f32),
    )(ap, s2, dp, wfc_row, b2, bfc)


def kernel(x, edge_index, W1, b1, W2, b2, Wfc, bfc):
    src = edge_index[0].astype(jnp.int32)
    dst = edge_index[1].astype(jnp.int32)
    pad = jnp.arange(EP - E, dtype=jnp.int32)
    srcp = jnp.concatenate([src, pad % N]).reshape(EP // CHUNK, CHUNK)
    dstp = jnp.concatenate([dst, N + pad % (NP - N)]).reshape(EP // CHUNK, CHUNK)

    ones16 = jnp.ones((CHUNK, 16), _f32)
    z16 = jnp.zeros((ROWS_PT, 16), _f32)
    z64 = jnp.zeros((ROWS_PT, 64), _f32)
    z32 = jnp.zeros((ROWS_PT, 32), _f32)

    dp = _deg_sc(dstp, ones16, z16)
    y1, s1 = _dense1(x, W1, dp)
    a1 = _agg64(y1, srcp, dstp, z64)
    y2, s2 = _dense2(a1, s1, dp, W2, b1.reshape(1, 64))
    a2 = _agg32(y2, srcp, dstp, z32)
    return _dense3(a2, s2, dp, Wfc.reshape(1, 32),
                   b2.reshape(1, 32), bfc.reshape(1, 1))


# speedup vs baseline: 26.7871x; 1.0790x over previous
import functools

import jax
import jax.numpy as jnp
from jax import lax
from jax.experimental import pallas as pl
from jax.experimental.pallas import tpu as pltpu
from jax.experimental.pallas import tpu_sc as plsc

N = 10000
E = 160000
NP = 10240
EP = 163840
CHUNK = 128
CPT = (EP // 32) // CHUNK
ROWS_PT = NP // 16

_mesh = plsc.VectorSubcoreMesh(core_axis_name="c", subcore_axis_name="s")
_f32 = jnp.float32
_sc_params = pltpu.CompilerParams(use_tc_tiling_on_sc=False)


@functools.partial(
    pl.kernel,
    out_type=jax.ShapeDtypeStruct((2, NP, 16), _f32),
    mesh=_mesh,
    scratch_types=[
        pltpu.VMEM((CPT, CHUNK), jnp.int32),
        pltpu.VMEM((CHUNK, 16), _f32),
        pltpu.VMEM_SHARED((NP, 16), _f32),
        pltpu.SemaphoreType.DMA,
    ],
    compiler_params=_sc_params,
)
def _deg_sc(dst_hbm, ones_hbm, zeros_hbm, out_hbm, idx_v, ones_v, acc, sem):
    cid = lax.axis_index("c")
    sid = lax.axis_index("s")
    tid = cid * 16 + sid
    pltpu.sync_copy(dst_hbm.at[pl.ds(tid * CPT, CPT)], idx_v)
    pltpu.sync_copy(ones_hbm, ones_v)
    pltpu.sync_copy(zeros_hbm, acc.at[pl.ds(sid * ROWS_PT, ROWS_PT)])
    plsc.subcore_barrier()

    @pl.loop(0, CPT, step=8)
    def _(j):
        for k in range(8):
            pltpu.async_copy(ones_v, acc.at[idx_v.at[j + k]], sem, add=True)
        for k in range(8):
            pltpu.make_async_copy(ones_v, acc.at[idx_v.at[j + k]], sem).wait()

    plsc.subcore_barrier()
    pltpu.sync_copy(
        acc.at[pl.ds(sid * ROWS_PT, ROWS_PT)],
        out_hbm.at[cid, pl.ds(sid * ROWS_PT, ROWS_PT)],
    )


def _make_agg(d):
    @functools.partial(
        pl.kernel,
        out_type=jax.ShapeDtypeStruct((2, NP, d), _f32),
        mesh=_mesh,
        scratch_types=[
            pltpu.VMEM((CPT, CHUNK), jnp.int32),
            pltpu.VMEM((CPT, CHUNK), jnp.int32),
            [pltpu.VMEM((CHUNK, d), _f32)] * 8,
            [pltpu.SemaphoreType.DMA] * 8,
            [pltpu.SemaphoreType.DMA] * 8,
            pltpu.VMEM_SHARED((NP, d), _f32),
        ],
        compiler_params=_sc_params,
    )
    def _agg(y_hbm, src_hbm, dst_hbm, zeros_hbm, out_hbm,
             srcv, dstv, bufs, gsems, ssems, acc):
        cid = lax.axis_index("c")
        sid = lax.axis_index("s")
        tid = cid * 16 + sid
        pltpu.sync_copy(src_hbm.at[pl.ds(tid * CPT, CPT)], srcv)
        pltpu.sync_copy(dst_hbm.at[pl.ds(tid * CPT, CPT)], dstv)
        pltpu.sync_copy(zeros_hbm, acc.at[pl.ds(sid * ROWS_PT, ROWS_PT)])
        plsc.subcore_barrier()

        for c in range(4):
            pltpu.async_copy(y_hbm.at[srcv.at[c]], bufs[c], gsems[c])

        @pl.loop(0, CPT, step=8)
        def _(j):
            for k in range(8):
                b = k % 8
                pltpu.make_async_copy(y_hbm.at[srcv.at[j + k]],
                                      bufs[b], gsems[b]).wait()
                pltpu.async_copy(bufs[b], acc.at[dstv.at[j + k]],
                                 ssems[b], add=True)
                bn = (k + 4) % 8

                @pl.when(j + k + 4 < CPT)
                def _():
                    @pl.when(j + k >= 4)
                    def _():
                        pltpu.make_async_copy(
                            bufs[bn], acc.at[dstv.at[j + k - 4]],
                            ssems[bn]).wait()

                    pltpu.async_copy(y_hbm.at[srcv.at[j + k + 4]],
                                     bufs[bn], gsems[bn])

        for c in range(CPT - 8, CPT):
            b = c % 8
            pltpu.make_async_copy(bufs[b], acc.at[dstv.at[c]],
                                  ssems[b]).wait()

        plsc.subcore_barrier()
        pltpu.sync_copy(
            acc.at[pl.ds(sid * ROWS_PT, ROWS_PT)],
            out_hbm.at[cid, pl.ds(sid * ROWS_PT, ROWS_PT)],
        )

    return _agg


_agg64 = _make_agg(64)
_agg32 = _make_agg(32)


_BLK = 2048
_GRID = NP // _BLK


def _dis_invdeg(dp_ref):
    deg = 1.0 + dp_ref[0, :, 0:1] + dp_ref[1, :, 0:1]
    dis = 1.0 / jnp.sqrt(deg)
    return dis, dis * dis


def _dense1_body(x_ref, w_ref, dp_ref, y_ref, s_ref):
    xw = jnp.dot(x_ref[...], w_ref[...], preferred_element_type=_f32)
    dis, invd = _dis_invdeg(dp_ref)
    y_ref[...] = xw * dis
    s_ref[...] = xw * invd


def _dense1(x, w1, dp):
    return pl.pallas_call(
        _dense1_body,
        grid=(_GRID,),
        in_specs=[
            pl.BlockSpec((_BLK, 256), lambda i: (i, 0)),
            pl.BlockSpec((256, 64), lambda i: (0, 0)),
            pl.BlockSpec((2, _BLK, 16), lambda i: (0, i, 0)),
        ],
        out_specs=[
            pl.BlockSpec((_BLK, 64), lambda i: (i, 0)),
            pl.BlockSpec((_BLK, 64), lambda i: (i, 0)),
        ],
        out_shape=[
            jax.ShapeDtypeStruct((N, 64), _f32),
            jax.ShapeDtypeStruct((N, 64), _f32),
        ],
    )(x, w1, dp)


def _dense2_body(ap_ref, s1_ref, dp_ref, w_ref, b_ref, y_ref, s_ref):
    dis, invd = _dis_invdeg(dp_ref)
    agg = ap_ref[0] + ap_ref[1]
    h = jnp.maximum(agg * dis + s1_ref[...] + b_ref[...], 0.0)
    xw = jnp.dot(h, w_ref[...], preferred_element_type=_f32)
    y_ref[...] = xw * dis
    s_ref[...] = xw * invd


def _dense2(ap, s1, dp, w2, b1):
    return pl.pallas_call(
        _dense2_body,
        grid=(_GRID,),
        in_specs=[
            pl.BlockSpec((2, _BLK, 64), lambda i: (0, i, 0)),
            pl.BlockSpec((_BLK, 64), lambda i: (i, 0)),
            pl.BlockSpec((2, _BLK, 16), lambda i: (0, i, 0)),
            pl.BlockSpec((64, 32), lambda i: (0, 0)),
            pl.BlockSpec((1, 64), lambda i: (0, 0)),
        ],
        out_specs=[
            pl.BlockSpec((_BLK, 32), lambda i: (i, 0)),
            pl.BlockSpec((_BLK, 32), lambda i: (i, 0)),
        ],
        out_shape=[
            jax.ShapeDtypeStruct((N, 32), _f32),
            jax.ShapeDtypeStruct((N, 32), _f32),
        ],
    )(ap, s1, dp, w2, b1)


def _dense3_body(ap_ref, s2_ref, dp_ref, wfc_ref, b2_ref, bfc_ref, o_ref):
    dis, _ = _dis_invdeg(dp_ref)
    agg = ap_ref[0] + ap_ref[1]
    h = jnp.maximum(agg * dis + s2_ref[...] + b2_ref[...], 0.0)
    o_ref[...] = jnp.sum(h * wfc_ref[...], axis=1, keepdims=True) + bfc_ref[...]


def _dense3(ap, s2, dp, wfc_row, b2, bfc):
    return pl.pallas_call(
        _dense3_body,
        grid=(_GRID,),
        in_specs=[
            pl.BlockSpec((2, _BLK, 32), lambda i: (0, i, 0)),
            pl.BlockSpec((_BLK, 32), lambda i: (i, 0)),
            pl.BlockSpec((2, _BLK, 16), lambda i: (0, i, 0)),
            pl.BlockSpec((1, 32), lambda i: (0, 0)),
            pl.BlockSpec((1, 32), lambda i: (0, 0)),
            pl.BlockSpec((1, 1), lambda i: (0, 0)),
        ],
        out_specs=pl.BlockSpec((_BLK, 1), lambda i: (i, 0)),
        out_shape=jax.ShapeDtypeStruct((N, 1), _f32),
    )(ap, s2, dp, wfc_row, b2, bfc)


def kernel(x, edge_index, W1, b1, W2, b2, Wfc, bfc):
    src = edge_index[0].astype(jnp.int32)
    dst = edge_index[1].astype(jnp.int32)
    pad = jnp.arange(EP - E, dtype=jnp.int32)
    srcp = jnp.concatenate([src, pad % N]).reshape(EP // CHUNK, CHUNK)
    dstp = jnp.concatenate([dst, N + pad % (NP - N)]).reshape(EP // CHUNK, CHUNK)

    ones16 = jnp.ones((CHUNK, 16), _f32)
    z16 = jnp.zeros((ROWS_PT, 16), _f32)
    z64 = jnp.zeros((ROWS_PT, 64), _f32)
    z32 = jnp.zeros((ROWS_PT, 32), _f32)

    dp = _deg_sc(dstp, ones16, z16)
    y1, s1 = _dense1(x, W1, dp)
    a1 = _agg64(y1, srcp, dstp, z64)
    y2, s2 = _dense2(a1, s1, dp, W2, b1.reshape(1, 64))
    a2 = _agg32(y2, srcp, dstp, z32)
    return _dense3(a2, s2, dp, Wfc.reshape(1, 32),
                   b2.reshape(1, 32), bfc.reshape(1, 1))
